# baseline (speedup 1.0000x reference)
import functools

import jax
import jax.numpy as jnp
from jax import lax
from jax.experimental import pallas as pl
from jax.experimental.pallas import tpu as pltpu
from jax.experimental.pallas import tpu_sc as plsc

N = 10000
E = 320000
HID = 128
DEPTH = 5
NG = 64
SGN = 230

NC = 2
NS = 16
NW = NC * NS
CHUNK = 128
CPW = 80
EPAD = NW * CPW * CHUNK
NCHUNK = EPAD // CHUNK
NCH = NCHUNK // 2
CPH = NCH // NW
EH = NCH * CHUNK
RS = 640
RS_LAST = N - (NS - 1) * RS

MBLK = 1280
MGRID = EPAD // MBLK
MGRID_H = EH // MBLK
NFULL1 = (E - EH) // MBLK

def _sc_mesh():
    return plsc.VectorSubcoreMesh(
        core_axis_name="c", subcore_axis_name="s", num_cores=NC, num_subcores=NS)


def _ln(t, g, b):
    m = jnp.mean(t, axis=-1, keepdims=True)
    v = jnp.mean((t - m) ** 2, axis=-1, keepdims=True)
    return (t - m) / jnp.sqrt(v + 1e-5) * g + b


def _enc_node_body(x_ref, w_ref, b_ref, g_ref, beta_ref, o_ref):
    x = x_ref[...]
    w = w_ref[...]
    acc = b_ref[...]
    for k in range(4):
        acc = acc + x[:, k:k + 1] * w[k:k + 1, :]
    o_ref[...] = jnp.maximum(_ln(acc, g_ref[...], beta_ref[...]), 0.0)


def _encode_nodes(x, w, b, g, beta):
    return pl.pallas_call(
        _enc_node_body,
        out_shape=jax.ShapeDtypeStruct((N, HID), jnp.float32),
    )(x, w, b, g, beta)


def _enc_edge_body(ea_ref, w_ref, b_ref, g_ref, beta_ref, o_ref):
    ea = ea_ref[...]
    w = w_ref[...]
    acc = b_ref[...]
    for k in range(3):
        acc = acc + ea[:, k:k + 1] * w[k:k + 1, :]
    o_ref[...] = jnp.maximum(_ln(acc, g_ref[...], beta_ref[...]), 0.0)


def _encode_edges(ea_pad, w, b, g, beta):
    return pl.pallas_call(
        _enc_edge_body,
        grid=(MGRID,),
        in_specs=[
            pl.BlockSpec((MBLK, 3), lambda i: (i, 0)),
            pl.BlockSpec((3, HID), lambda i: (0, 0)),
            pl.BlockSpec((1, HID), lambda i: (0, 0)),
            pl.BlockSpec((1, HID), lambda i: (0, 0)),
            pl.BlockSpec((1, HID), lambda i: (0, 0)),
        ],
        out_specs=pl.BlockSpec((MBLK, HID), lambda i: (i, 0)),
        out_shape=jax.ShapeDtypeStruct((EPAD, HID), jnp.float32),
    )(ea_pad, w, b, g, beta)


def _proj_body(h_ref, wa_ref, wb_ref, h1_ref, h2_ref):
    h = h_ref[...]
    h1_ref[...] = jnp.dot(h, wa_ref[...], preferred_element_type=jnp.float32)
    h2_ref[...] = jnp.dot(h, wb_ref[...], preferred_element_type=jnp.float32)


def _project(h, wa, wb):
    return pl.pallas_call(
        _proj_body,
        out_shape=(jax.ShapeDtypeStruct((N, HID), jnp.float32),
                   jax.ShapeDtypeStruct((N, HID), jnp.float32)),
    )(h, wa, wb)


def _make_msg_body(nfull):
    def _msg_body(g1_ref, g2_ref, ea_ref, wc_ref, b_ref, g_ref, beta_ref, o_ref):
        def compute():
            s = (g1_ref[...] + g2_ref[...]
                 + jnp.dot(ea_ref[...], wc_ref[...], preferred_element_type=jnp.float32)
                 + b_ref[...])
            o_ref[...] = jnp.maximum(_ln(s, g_ref[...], beta_ref[...]), 0.0)

        if nfull >= MGRID_H:
            compute()
        else:
            pid = pl.program_id(0)

            @pl.when(pid < nfull)
            def _():
                compute()

            @pl.when(pid >= nfull)
            def _():
                o_ref[...] = jnp.zeros_like(o_ref)

    return _msg_body


def _make_message(nfull, blk_ofs):
    return pl.pallas_call(
        _make_msg_body(nfull),
        grid=(MGRID_H,),
        in_specs=[
            pl.BlockSpec((MBLK, HID), lambda i: (i + blk_ofs, 0)),
            pl.BlockSpec((MBLK, HID), lambda i: (i + blk_ofs, 0)),
            pl.BlockSpec((MBLK, HID), lambda i: (i + blk_ofs, 0)),
            pl.BlockSpec((HID, HID), lambda i: (0, 0)),
            pl.BlockSpec((1, HID), lambda i: (0, 0)),
            pl.BlockSpec((1, HID), lambda i: (0, 0)),
            pl.BlockSpec((1, HID), lambda i: (0, 0)),
        ],
        out_specs=pl.BlockSpec((MBLK, HID), lambda i: (i, 0)),
        out_shape=jax.ShapeDtypeStruct((EH, HID), jnp.float32),
    )


_MESSAGE_A = _make_message(MGRID_H, 0)
_MESSAGE_B = _make_message(NFULL1, MGRID_H)


def _upd_body(h_ref, agga_ref, aggb_ref, g_ref, beta_ref, o_ref):
    t = (h_ref[...] + agga_ref[0] + agga_ref[1]
         + aggb_ref[0] + aggb_ref[1])
    o_ref[...] = jnp.maximum(_ln(t, g_ref[...], beta_ref[...]), 0.0)


def _update(h, agga, aggb, g, beta):
    return pl.pallas_call(
        _upd_body,
        out_shape=jax.ShapeDtypeStruct((N, HID), jnp.float32),
    )(h, agga, aggb, g, beta)


def _readout_body(h_ref, attw_ref, attb_ref, batch_ref, sg_ref, sgemb_ref,
                  w1t_ref, w1b_ref, b1_ref, w2_ref, b2_ref, o_ref):
    h = h_ref[...]
    s = jnp.dot(h, attw_ref[...], preferred_element_type=jnp.float32) + attb_ref[...]
    m = jnp.max(s)
    e = jnp.exp(s - m)
    alpha = e / jnp.sum(e)
    gi = lax.broadcasted_iota(jnp.int32, (NG, N), 0)
    oh = (gi == batch_ref[...]).astype(jnp.float32)
    pooled = jnp.dot(oh, alpha * h, preferred_element_type=jnp.float32)
    sgi = lax.broadcasted_iota(jnp.int32, (NG, SGN), 1)
    ohsg = (sgi == sg_ref[...]).astype(jnp.float32)
    sg = jnp.dot(ohsg, sgemb_ref[...], preferred_element_type=jnp.float32)
    hh = (jnp.dot(pooled, w1t_ref[...], preferred_element_type=jnp.float32)
          + jnp.dot(sg, w1b_ref[...], preferred_element_type=jnp.float32)
          + b1_ref[...])
    hh = jnp.maximum(hh, 0.0)
    o_ref[...] = jnp.dot(hh, w2_ref[...], preferred_element_type=jnp.float32) + b2_ref[...]


def _readout(h, attw, attb, batch2d, sg2d, sgemb, w1t, w1b, b1, w2, b2):
    return pl.pallas_call(
        _readout_body,
        out_shape=jax.ShapeDtypeStruct((NG, HID), jnp.float32),
    )(h, attw, attb, batch2d, sg2d, sgemb, w1t, w1b, b1, w2, b2)


SUP = 2
NSUPW = CPH // SUP


EPSF = EPAD // NS
PHE = EPSF // 2
CPS = PHE // CHUNK


def _gather_body(h1, h2, ridx, cidx, g1, g2, idx_v, bufa, bufb,
                 gsa, gsb, ssa, ssb, spm_tab):
    c = lax.axis_index("c")
    s = lax.axis_index("s")
    bufs = (bufa, bufb)
    gs = (gsa, gsb)
    ss = (ssa, ssb)

    def run(tab_hbm, idx_hbm, out_hbm):
        base = s * RS

        @pl.when(s < NS - 1)
        def _():
            pltpu.sync_copy(tab_hbm.at[pl.ds(base, RS)], spm_tab.at[pl.ds(base, RS)])

        @pl.when(s == NS - 1)
        def _():
            pltpu.sync_copy(tab_hbm.at[pl.ds((NS - 1) * RS, RS_LAST)],
                            spm_tab.at[pl.ds((NS - 1) * RS, RS_LAST)])

        pltpu.sync_copy(idx_hbm.at[pl.ds(s * EPSF, PHE)], idx_v)
        plsc.subcore_barrier()

        for ph in range(2):
            out_base = s * EPSF + ph * PHE

            def fire_gather(j, p):
                pltpu.async_copy(spm_tab.at[idx_v.at[pl.ds(j * CHUNK, CHUNK)]],
                                 bufs[p], gs[p])

            def wait_gather(j, p):
                pltpu.make_async_copy(spm_tab.at[idx_v.at[pl.ds(j * CHUNK, CHUNK)]],
                                      bufs[p], gs[p]).wait()

            def wait_store(p):
                pltpu.make_async_copy(bufs[p], out_hbm.at[pl.ds(0, CHUNK)], ss[p]).wait()

            fire_gather(0, 0)

            def body(jj, _):
                for p in range(2):
                    j = jj * 2 + p
                    q = 1 - p

                    @pl.when(j + 1 < CPS)
                    def _():
                        @pl.when(j >= 1)
                        def _():
                            wait_store(q)
                        fire_gather(j + 1, q)

                    wait_gather(j, p)
                    pltpu.async_copy(bufs[p],
                                     out_hbm.at[pl.ds(out_base + j * CHUNK, CHUNK)],
                                     ss[p])
                return 0

            lax.fori_loop(0, CPS // 2, body, 0)
            wait_store(0)
            wait_store(1)
            if ph == 0:
                pltpu.sync_copy(idx_hbm.at[pl.ds(s * EPSF + PHE, PHE)], idx_v)

    @pl.when(c == 0)
    def _():
        run(h1, ridx, g1)

    @pl.when(c == 1)
    def _():
        run(h2, cidx, g2)


_GATHER_K = None


def _gather_sc(h1, h2, ridx_flat, cidx_flat):
    global _GATHER_K
    if _GATHER_K is None:
        _GATHER_K = pl.kernel(
            _gather_body,
            out_type=(jax.ShapeDtypeStruct((EPAD, HID), jnp.float32),
                      jax.ShapeDtypeStruct((EPAD, HID), jnp.float32)),
            mesh=_sc_mesh(),
            scratch_types=(
                [pltpu.VMEM((PHE,), jnp.int32)]
                + [pltpu.VMEM((CHUNK, HID), jnp.float32)] * 2
                + [pltpu.SemaphoreType.DMA] * 4
                + [pltpu.VMEM_SHARED((N, HID), jnp.float32)]
            ),
        )
    return _GATHER_K(h1, h2, ridx_flat, cidx_flat)


def _scatter_body(msg, ridx, zeros_n, out, ridx_v, mbufa, mbufb,
                  lsa, lsb, asa, asb, acc):
    c = lax.axis_index("c")
    s = lax.axis_index("s")
    wid = s * NC + c
    pltpu.sync_copy(ridx.at[pl.ds(wid * CPH, CPH)], ridx_v)
    base = s * RS

    @pl.when(s < NS - 1)
    def _():
        pltpu.sync_copy(zeros_n.at[pl.ds(base, RS)], acc.at[pl.ds(base, RS)])

    @pl.when(s == NS - 1)
    def _():
        pltpu.sync_copy(zeros_n.at[pl.ds((NS - 1) * RS, RS_LAST)],
                        acc.at[pl.ds((NS - 1) * RS, RS_LAST)])

    plsc.subcore_barrier()

    mbufs = (mbufa, mbufb)
    lsem = (lsa, lsb)
    asem = (asa, asb)

    def fire_load(j, p):
        chunk = wid * CPH + j
        pltpu.async_copy(msg.at[pl.ds(chunk * CHUNK, CHUNK)], mbufs[p], lsem[p])

    def wait_add(p):
        pltpu.make_async_copy(mbufs[p], acc.at[pl.ds(0, CHUNK)], asem[p]).wait()

    fire_load(0, 0)

    def body(jj, _):
        for p in range(2):
            j = jj * 2 + p
            q = 1 - p

            @pl.when(j + 1 < CPH)
            def _():
                @pl.when(j >= 1)
                def _():
                    wait_add(q)
                fire_load(j + 1, q)

            pltpu.make_async_copy(msg.at[pl.ds(0, CHUNK)], mbufs[p], lsem[p]).wait()
            pltpu.async_copy(mbufs[p], acc.at[ridx_v.at[j]], asem[p], add=True)
        return 0

    lax.fori_loop(0, CPH // 2, body, 0)
    wait_add(0)
    wait_add(1)
    plsc.subcore_barrier()

    @pl.when(s < NS - 1)
    def _():
        pltpu.sync_copy(acc.at[pl.ds(base, RS)], out.at[c, pl.ds(base, RS)])

    @pl.when(s == NS - 1)
    def _():
        pltpu.sync_copy(acc.at[pl.ds((NS - 1) * RS, RS_LAST)],
                        out.at[c, pl.ds((NS - 1) * RS, RS_LAST)])


_SCATTER_K = None


def _scatter_sc(msg, ridx, zeros_n):
    global _SCATTER_K
    if _SCATTER_K is None:
        _SCATTER_K = pl.kernel(
            _scatter_body,
            out_type=jax.ShapeDtypeStruct((NC, N, HID), jnp.float32),
            mesh=_sc_mesh(),
            scratch_types=(
                [pltpu.VMEM((CPH, CHUNK), jnp.int32)]
                + [pltpu.VMEM((CHUNK, HID), jnp.float32)] * 2
                + [pltpu.SemaphoreType.DMA] * 4
                + [pltpu.VMEM_SHARED((N, HID), jnp.float32)]
            ),
        )
    return _SCATTER_K(msg, ridx, zeros_n)


def kernel(params, x, edge_attr, edge_index, batch, space_group):
    p = params
    f32 = jnp.float32
    row = edge_index[0]
    col = edge_index[1]
    pad_idx = (jnp.arange(EPAD - E, dtype=jnp.int32) % N)
    row_p = jnp.concatenate([row, pad_idx]).reshape(NCHUNK, CHUNK)
    col_p = jnp.concatenate([col, pad_idx]).reshape(NCHUNK, CHUNK)
    ea_pad = jnp.concatenate([edge_attr, jnp.zeros((EPAD - E, 3), f32)])
    zeros_n = jnp.zeros((N, HID), f32)

    def v2(a):
        return a.reshape(1, -1)

    h = _encode_nodes(x, p['node_w'], v2(p['node_b']), v2(p['node_g']), v2(p['node_beta']))
    ea_enc = _encode_edges(ea_pad, p['edge_w'], v2(p['edge_b']), v2(p['edge_g']), v2(p['edge_beta']))

    row_a, row_b = row_p[:NCH], row_p[NCH:]
    row_f = row_p.reshape(-1)
    col_f = col_p.reshape(-1)

    for l in range(DEPTH):
        W = p['mp_w'][l]
        wa, wb, wc = W[:HID], W[HID:2 * HID], W[2 * HID:]
        mb, mg, mbeta = v2(p['mp_b'][l]), v2(p['mp_g'][l]), v2(p['mp_beta'][l])
        h1, h2 = _project(h, wa, wb)
        g1, g2 = _gather_sc(h1, h2, row_f, col_f)
        msg_a = _MESSAGE_A(g1, g2, ea_enc, wc, mb, mg, mbeta)
        agg_a = _scatter_sc(msg_a, row_a, zeros_n)
        msg_b = _MESSAGE_B(g1, g2, ea_enc, wc, mb, mg, mbeta)
        msg_b2, _ = lax.optimization_barrier((msg_b, agg_a))
        agg_b = _scatter_sc(msg_b2, row_b, zeros_n)
        h = _update(h, agg_a, agg_b, v2(p['ln_g'][l]), v2(p['ln_b'][l]))

    heads = ['e', 'st', 'cs', 'mt']
    odims = [1, 3, 7, 3]
    w1t = jnp.concatenate([p[k + '_w1'][:HID] for k in heads], axis=1)
    w1b = jnp.concatenate([p[k + '_w1'][HID:] for k in heads], axis=1)
    b1 = jnp.concatenate([p[k + '_b1'] for k in heads]).reshape(1, 4 * HID)
    w2 = jnp.zeros((4 * HID, HID), f32)
    b2 = jnp.zeros((1, HID), f32)
    off = 0
    for i, k in enumerate(heads):
        w2 = w2.at[i * HID:(i + 1) * HID, off:off + odims[i]].set(p[k + '_w2'])
        b2 = b2.at[0, off:off + odims[i]].set(p[k + '_b2'])
        off += odims[i]

    out = _readout(h, p['att_w'], p['att_b'].reshape(1, 1),
                   batch.reshape(1, N), space_group.reshape(NG, 1).astype(jnp.int32),
                   p['sg_emb'], w1t, w1b, b1, w2, b2)
    return (out[:, :1], out[:, 1:4], out[:, 4:11], out[:, 11:14])


# speedup vs baseline: 1.0349x; 1.0349x over previous
import functools

import jax
import jax.numpy as jnp
from jax import lax
from jax.experimental import pallas as pl
from jax.experimental.pallas import tpu as pltpu
from jax.experimental.pallas import tpu_sc as plsc

N = 10000
E = 320000
HID = 128
DEPTH = 5
NG = 64
SGN = 230

NC = 2
NS = 16
NW = NC * NS
CHUNK = 128
CPW = 80
EPAD = NW * CPW * CHUNK
NCHUNK = EPAD // CHUNK
NCH = NCHUNK // 2
CPH = NCH // NW
EH = NCH * CHUNK
RS = 640
RS_LAST = N - (NS - 1) * RS

MBLK = 1280
MGRID = EPAD // MBLK
MGRID_H = EH // MBLK
NFULL1 = (E - EH) // MBLK

def _sc_mesh():
    return plsc.VectorSubcoreMesh(
        core_axis_name="c", subcore_axis_name="s", num_cores=NC, num_subcores=NS)


def _ln(t, g, b):
    m = jnp.mean(t, axis=-1, keepdims=True)
    v = jnp.mean((t - m) ** 2, axis=-1, keepdims=True)
    return (t - m) / jnp.sqrt(v + 1e-5) * g + b


def _enc_node_body(x_ref, w_ref, b_ref, g_ref, beta_ref, o_ref):
    x = x_ref[...]
    w = w_ref[...]
    acc = b_ref[...]
    for k in range(4):
        acc = acc + x[:, k:k + 1] * w[k:k + 1, :]
    o_ref[...] = jnp.maximum(_ln(acc, g_ref[...], beta_ref[...]), 0.0)


def _encode_nodes(x, w, b, g, beta):
    return pl.pallas_call(
        _enc_node_body,
        out_shape=jax.ShapeDtypeStruct((N, HID), jnp.float32),
    )(x, w, b, g, beta)


def _enc_edge_body(ea_ref, w_ref, b_ref, g_ref, beta_ref, o_ref):
    ea = ea_ref[...]
    w = w_ref[...]
    acc = b_ref[...]
    for k in range(3):
        acc = acc + ea[:, k:k + 1] * w[k:k + 1, :]
    o_ref[...] = jnp.maximum(_ln(acc, g_ref[...], beta_ref[...]), 0.0)


def _encode_edges(ea_pad, w, b, g, beta):
    return pl.pallas_call(
        _enc_edge_body,
        grid=(MGRID,),
        in_specs=[
            pl.BlockSpec((MBLK, 3), lambda i: (i, 0)),
            pl.BlockSpec((3, HID), lambda i: (0, 0)),
            pl.BlockSpec((1, HID), lambda i: (0, 0)),
            pl.BlockSpec((1, HID), lambda i: (0, 0)),
            pl.BlockSpec((1, HID), lambda i: (0, 0)),
        ],
        out_specs=pl.BlockSpec((MBLK, HID), lambda i: (i, 0)),
        out_shape=jax.ShapeDtypeStruct((EPAD, HID), jnp.float32),
    )(ea_pad, w, b, g, beta)


def _proj_body(h_ref, wa_ref, wb_ref, h1_ref, h2_ref):
    h = h_ref[...]
    h1_ref[...] = jnp.dot(h, wa_ref[...], preferred_element_type=jnp.float32)
    h2_ref[...] = jnp.dot(h, wb_ref[...], preferred_element_type=jnp.float32)


def _project(h, wa, wb):
    return pl.pallas_call(
        _proj_body,
        out_shape=(jax.ShapeDtypeStruct((N, HID), jnp.float32),
                   jax.ShapeDtypeStruct((N, HID), jnp.float32)),
    )(h, wa, wb)


def _make_msg_body(nfull):
    def _msg_body(g1_ref, g2_ref, ea_ref, wc_ref, b_ref, g_ref, beta_ref, o_ref):
        def compute():
            s = (g1_ref[...] + g2_ref[...]
                 + jnp.dot(ea_ref[...], wc_ref[...], preferred_element_type=jnp.float32)
                 + b_ref[...])
            o_ref[...] = jnp.maximum(_ln(s, g_ref[...], beta_ref[...]), 0.0)

        if nfull >= MGRID_H:
            compute()
        else:
            pid = pl.program_id(0)

            @pl.when(pid < nfull)
            def _():
                compute()

            @pl.when(pid >= nfull)
            def _():
                o_ref[...] = jnp.zeros_like(o_ref)

    return _msg_body


def _make_message(nfull, blk_ofs):
    return pl.pallas_call(
        _make_msg_body(nfull),
        grid=(MGRID_H,),
        in_specs=[
            pl.BlockSpec((MBLK, HID), lambda i: (i, 0)),
            pl.BlockSpec((MBLK, HID), lambda i: (i, 0)),
            pl.BlockSpec((MBLK, HID), lambda i: (i + blk_ofs, 0)),
            pl.BlockSpec((HID, HID), lambda i: (0, 0)),
            pl.BlockSpec((1, HID), lambda i: (0, 0)),
            pl.BlockSpec((1, HID), lambda i: (0, 0)),
            pl.BlockSpec((1, HID), lambda i: (0, 0)),
        ],
        out_specs=pl.BlockSpec((MBLK, HID), lambda i: (i, 0)),
        out_shape=jax.ShapeDtypeStruct((EH, HID), jnp.float32),
    )


_MESSAGE_A = _make_message(MGRID_H, 0)
_MESSAGE_B = _make_message(NFULL1, MGRID_H)


def _upd_body(h_ref, agga_ref, aggb_ref, g_ref, beta_ref, o_ref):
    t = (h_ref[...] + agga_ref[0] + agga_ref[1]
         + aggb_ref[0] + aggb_ref[1])
    o_ref[...] = jnp.maximum(_ln(t, g_ref[...], beta_ref[...]), 0.0)


def _update(h, agga, aggb, g, beta):
    return pl.pallas_call(
        _upd_body,
        out_shape=jax.ShapeDtypeStruct((N, HID), jnp.float32),
    )(h, agga, aggb, g, beta)


def _readout_body(h_ref, attw_ref, attb_ref, batch_ref, sg_ref, sgemb_ref,
                  w1t_ref, w1b_ref, b1_ref, w2_ref, b2_ref, o_ref):
    h = h_ref[...]
    s = jnp.dot(h, attw_ref[...], preferred_element_type=jnp.float32) + attb_ref[...]
    m = jnp.max(s)
    e = jnp.exp(s - m)
    alpha = e / jnp.sum(e)
    gi = lax.broadcasted_iota(jnp.int32, (NG, N), 0)
    oh = (gi == batch_ref[...]).astype(jnp.float32)
    pooled = jnp.dot(oh, alpha * h, preferred_element_type=jnp.float32)
    sgi = lax.broadcasted_iota(jnp.int32, (NG, SGN), 1)
    ohsg = (sgi == sg_ref[...]).astype(jnp.float32)
    sg = jnp.dot(ohsg, sgemb_ref[...], preferred_element_type=jnp.float32)
    hh = (jnp.dot(pooled, w1t_ref[...], preferred_element_type=jnp.float32)
          + jnp.dot(sg, w1b_ref[...], preferred_element_type=jnp.float32)
          + b1_ref[...])
    hh = jnp.maximum(hh, 0.0)
    o_ref[...] = jnp.dot(hh, w2_ref[...], preferred_element_type=jnp.float32) + b2_ref[...]


def _readout(h, attw, attb, batch2d, sg2d, sgemb, w1t, w1b, b1, w2, b2):
    return pl.pallas_call(
        _readout_body,
        out_shape=jax.ShapeDtypeStruct((NG, HID), jnp.float32),
    )(h, attw, attb, batch2d, sg2d, sgemb, w1t, w1b, b1, w2, b2)


SUP = 2
NSUPW = CPH // SUP


EPS = EH // NS
CPS = EPS // CHUNK


def _gather_body(h1, h2, ridx, cidx, g1, g2, idx_v, bufa, bufb,
                 gsa, gsb, ssa, ssb, spm_tab):
    c = lax.axis_index("c")
    s = lax.axis_index("s")
    bufs = (bufa, bufb)
    gs = (gsa, gsb)
    ss = (ssa, ssb)

    def run(tab_hbm, idx_hbm, out_hbm):
        base = s * RS

        @pl.when(s < NS - 1)
        def _():
            pltpu.sync_copy(tab_hbm.at[pl.ds(base, RS)], spm_tab.at[pl.ds(base, RS)])

        @pl.when(s == NS - 1)
        def _():
            pltpu.sync_copy(tab_hbm.at[pl.ds((NS - 1) * RS, RS_LAST)],
                            spm_tab.at[pl.ds((NS - 1) * RS, RS_LAST)])

        pltpu.sync_copy(idx_hbm.at[pl.ds(s * EPS, EPS)], idx_v)
        plsc.subcore_barrier()

        def fire_gather(j, p):
            pltpu.async_copy(spm_tab.at[idx_v.at[pl.ds(j * CHUNK, CHUNK)]],
                             bufs[p], gs[p])

        def wait_gather(j, p):
            pltpu.make_async_copy(spm_tab.at[idx_v.at[pl.ds(j * CHUNK, CHUNK)]],
                                  bufs[p], gs[p]).wait()

        def wait_store(p):
            pltpu.make_async_copy(bufs[p], out_hbm.at[pl.ds(0, CHUNK)], ss[p]).wait()

        fire_gather(0, 0)

        def body(jj, _):
            for p in range(2):
                j = jj * 2 + p
                q = 1 - p

                @pl.when(j + 1 < CPS)
                def _():
                    @pl.when(j >= 1)
                    def _():
                        wait_store(q)
                    fire_gather(j + 1, q)

                wait_gather(j, p)
                pltpu.async_copy(bufs[p], out_hbm.at[pl.ds(s * EPS + j * CHUNK, CHUNK)],
                                 ss[p])
            return 0

        lax.fori_loop(0, CPS // 2, body, 0)
        wait_store(0)
        wait_store(1)

    @pl.when(c == 0)
    def _():
        run(h1, ridx, g1)

    @pl.when(c == 1)
    def _():
        run(h2, cidx, g2)


_GATHER_K = None


def _gather_sc(h1, h2, ridx_flat, cidx_flat):
    global _GATHER_K
    if _GATHER_K is None:
        _GATHER_K = pl.kernel(
            _gather_body,
            out_type=(jax.ShapeDtypeStruct((EH, HID), jnp.float32),
                      jax.ShapeDtypeStruct((EH, HID), jnp.float32)),
            mesh=_sc_mesh(),
            scratch_types=(
                [pltpu.VMEM((EPS,), jnp.int32)]
                + [pltpu.VMEM((CHUNK, HID), jnp.float32)] * 2
                + [pltpu.SemaphoreType.DMA] * 4
                + [pltpu.VMEM_SHARED((N, HID), jnp.float32)]
            ),
        )
    return _GATHER_K(h1, h2, ridx_flat, cidx_flat)


def _scatter_body(msg, ridx, zeros_n, out, ridx_v, mbufa, mbufb,
                  lsa, lsb, asa, asb, acc):
    c = lax.axis_index("c")
    s = lax.axis_index("s")
    wid = s * NC + c
    pltpu.sync_copy(ridx.at[pl.ds(wid * CPH, CPH)], ridx_v)
    base = s * RS

    @pl.when(s < NS - 1)
    def _():
        pltpu.sync_copy(zeros_n.at[pl.ds(base, RS)], acc.at[pl.ds(base, RS)])

    @pl.when(s == NS - 1)
    def _():
        pltpu.sync_copy(zeros_n.at[pl.ds((NS - 1) * RS, RS_LAST)],
                        acc.at[pl.ds((NS - 1) * RS, RS_LAST)])

    plsc.subcore_barrier()

    mbufs = (mbufa, mbufb)
    lsem = (lsa, lsb)
    asem = (asa, asb)

    def fire_load(j, p):
        chunk = wid * CPH + j
        pltpu.async_copy(msg.at[pl.ds(chunk * CHUNK, CHUNK)], mbufs[p], lsem[p])

    def wait_add(p):
        pltpu.make_async_copy(mbufs[p], acc.at[pl.ds(0, CHUNK)], asem[p]).wait()

    fire_load(0, 0)

    def body(jj, _):
        for p in range(2):
            j = jj * 2 + p
            q = 1 - p

            @pl.when(j + 1 < CPH)
            def _():
                @pl.when(j >= 1)
                def _():
                    wait_add(q)
                fire_load(j + 1, q)

            pltpu.make_async_copy(msg.at[pl.ds(0, CHUNK)], mbufs[p], lsem[p]).wait()
            pltpu.async_copy(mbufs[p], acc.at[ridx_v.at[j]], asem[p], add=True)
        return 0

    lax.fori_loop(0, CPH // 2, body, 0)
    wait_add(0)
    wait_add(1)
    plsc.subcore_barrier()

    @pl.when(s < NS - 1)
    def _():
        pltpu.sync_copy(acc.at[pl.ds(base, RS)], out.at[c, pl.ds(base, RS)])

    @pl.when(s == NS - 1)
    def _():
        pltpu.sync_copy(acc.at[pl.ds((NS - 1) * RS, RS_LAST)],
                        out.at[c, pl.ds((NS - 1) * RS, RS_LAST)])


_SCATTER_K = None


def _scatter_sc(msg, ridx, zeros_n):
    global _SCATTER_K
    if _SCATTER_K is None:
        _SCATTER_K = pl.kernel(
            _scatter_body,
            out_type=jax.ShapeDtypeStruct((NC, N, HID), jnp.float32),
            mesh=_sc_mesh(),
            scratch_types=(
                [pltpu.VMEM((CPH, CHUNK), jnp.int32)]
                + [pltpu.VMEM((CHUNK, HID), jnp.float32)] * 2
                + [pltpu.SemaphoreType.DMA] * 4
                + [pltpu.VMEM_SHARED((N, HID), jnp.float32)]
            ),
        )
    return _SCATTER_K(msg, ridx, zeros_n)


def kernel(params, x, edge_attr, edge_index, batch, space_group):
    p = params
    f32 = jnp.float32
    row = edge_index[0]
    col = edge_index[1]
    pad_idx = (jnp.arange(EPAD - E, dtype=jnp.int32) % N)
    row_p = jnp.concatenate([row, pad_idx]).reshape(NCHUNK, CHUNK)
    col_p = jnp.concatenate([col, pad_idx]).reshape(NCHUNK, CHUNK)
    ea_pad = jnp.concatenate([edge_attr, jnp.zeros((EPAD - E, 3), f32)])
    zeros_n = jnp.zeros((N, HID), f32)

    def v2(a):
        return a.reshape(1, -1)

    h = _encode_nodes(x, p['node_w'], v2(p['node_b']), v2(p['node_g']), v2(p['node_beta']))
    ea_enc = _encode_edges(ea_pad, p['edge_w'], v2(p['edge_b']), v2(p['edge_g']), v2(p['edge_beta']))

    row_a, row_b = row_p[:NCH], row_p[NCH:]
    row_fa, row_fb = row_a.reshape(-1), row_b.reshape(-1)
    col_fa = col_p[:NCH].reshape(-1)
    col_fb = col_p[NCH:].reshape(-1)

    for l in range(DEPTH):
        W = p['mp_w'][l]
        wa, wb, wc = W[:HID], W[HID:2 * HID], W[2 * HID:]
        mb, mg, mbeta = v2(p['mp_b'][l]), v2(p['mp_g'][l]), v2(p['mp_beta'][l])
        h1, h2 = _project(h, wa, wb)
        g1a, g2a = _gather_sc(h1, h2, row_fa, col_fa)
        row_fb2, _ = lax.optimization_barrier((row_fb, g1a))
        g1b, g2b = _gather_sc(h1, h2, row_fb2, col_fb)
        msg_a = _MESSAGE_A(g1a, g2a, ea_enc, wc, mb, mg, mbeta)
        msg_a2, _ = lax.optimization_barrier((msg_a, g1b))
        agg_a = _scatter_sc(msg_a2, row_a, zeros_n)
        msg_b = _MESSAGE_B(g1b, g2b, ea_enc, wc, mb, mg, mbeta)
        msg_b2, _ = lax.optimization_barrier((msg_b, agg_a))
        agg_b = _scatter_sc(msg_b2, row_b, zeros_n)
        h = _update(h, agg_a, agg_b, v2(p['ln_g'][l]), v2(p['ln_b'][l]))

    heads = ['e', 'st', 'cs', 'mt']
    odims = [1, 3, 7, 3]
    w1t = jnp.concatenate([p[k + '_w1'][:HID] for k in heads], axis=1)
    w1b = jnp.concatenate([p[k + '_w1'][HID:] for k in heads], axis=1)
    b1 = jnp.concatenate([p[k + '_b1'] for k in heads]).reshape(1, 4 * HID)
    w2 = jnp.zeros((4 * HID, HID), f32)
    b2 = jnp.zeros((1, HID), f32)
    off = 0
    for i, k in enumerate(heads):
        w2 = w2.at[i * HID:(i + 1) * HID, off:off + odims[i]].set(p[k + '_w2'])
        b2 = b2.at[0, off:off + odims[i]].set(p[k + '_b2'])
        off += odims[i]

    out = _readout(h, p['att_w'], p['att_b'].reshape(1, 1),
                   batch.reshape(1, N), space_group.reshape(NG, 1).astype(jnp.int32),
                   p['sg_emb'], w1t, w1b, b1, w2, b2)
    return (out[:, :1], out[:, 1:4], out[:, 4:11], out[:, 11:14])


# speedup vs baseline: 1.0413x; 1.0062x over previous
import functools

import jax
import jax.numpy as jnp
from jax import lax
from jax.experimental import pallas as pl
from jax.experimental.pallas import tpu as pltpu
from jax.experimental.pallas import tpu_sc as plsc

N = 10000
E = 320000
HID = 128
DEPTH = 5
NG = 64
SGN = 230

NC = 2
NS = 16
NW = NC * NS
CHUNK = 128
CPW = 80
EPAD = NW * CPW * CHUNK
NCHUNK = EPAD // CHUNK
NCH = NCHUNK // 2
CPH = NCH // NW
EH = NCH * CHUNK
RS = 640
RS_LAST = N - (NS - 1) * RS

MBLK = 1280
MGRID = EPAD // MBLK
MGRID_H = EH // MBLK
NFULL1 = (E - EH) // MBLK

def _sc_mesh():
    return plsc.VectorSubcoreMesh(
        core_axis_name="c", subcore_axis_name="s", num_cores=NC, num_subcores=NS)


def _ln(t, g, b):
    m = jnp.mean(t, axis=-1, keepdims=True)
    v = jnp.mean((t - m) ** 2, axis=-1, keepdims=True)
    return (t - m) / jnp.sqrt(v + 1e-5) * g + b


def _enc_node_body(x_ref, w_ref, b_ref, g_ref, beta_ref, o_ref):
    x = x_ref[...]
    w = w_ref[...]
    acc = b_ref[...]
    for k in range(4):
        acc = acc + x[:, k:k + 1] * w[k:k + 1, :]
    o_ref[...] = jnp.maximum(_ln(acc, g_ref[...], beta_ref[...]), 0.0)


def _encode_nodes(x, w, b, g, beta):
    return pl.pallas_call(
        _enc_node_body,
        out_shape=jax.ShapeDtypeStruct((N, HID), jnp.float32),
    )(x, w, b, g, beta)


def _enc_edge_body(ea_ref, w_ref, b_ref, g_ref, beta_ref, o_ref):
    ea = ea_ref[...]
    w = w_ref[...]
    acc = b_ref[...]
    for k in range(3):
        acc = acc + ea[:, k:k + 1] * w[k:k + 1, :]
    o_ref[...] = jnp.maximum(_ln(acc, g_ref[...], beta_ref[...]), 0.0)


def _encode_edges(ea_pad, w, b, g, beta):
    return pl.pallas_call(
        _enc_edge_body,
        grid=(MGRID,),
        in_specs=[
            pl.BlockSpec((MBLK, 3), lambda i: (i, 0)),
            pl.BlockSpec((3, HID), lambda i: (0, 0)),
            pl.BlockSpec((1, HID), lambda i: (0, 0)),
            pl.BlockSpec((1, HID), lambda i: (0, 0)),
            pl.BlockSpec((1, HID), lambda i: (0, 0)),
        ],
        out_specs=pl.BlockSpec((MBLK, HID), lambda i: (i, 0)),
        out_shape=jax.ShapeDtypeStruct((EPAD, HID), jnp.float32),
    )(ea_pad, w, b, g, beta)


def _proj_body(h_ref, wa_ref, wb_ref, h1_ref, h2_ref):
    h = h_ref[...]
    h1_ref[...] = jnp.dot(h, wa_ref[...], preferred_element_type=jnp.float32)
    h2_ref[...] = jnp.dot(h, wb_ref[...], preferred_element_type=jnp.float32)


def _project(h, wa, wb):
    return pl.pallas_call(
        _proj_body,
        out_shape=(jax.ShapeDtypeStruct((N, HID), jnp.float32),
                   jax.ShapeDtypeStruct((N, HID), jnp.float32)),
    )(h, wa, wb)


def _make_msg_body(nfull):
    def _msg_body(g1_ref, g2_ref, ea_ref, wc_ref, b_ref, g_ref, beta_ref, o_ref):
        def compute():
            s = (g1_ref[...] + g2_ref[...]
                 + jnp.dot(ea_ref[...], wc_ref[...], preferred_element_type=jnp.float32)
                 + b_ref[...])
            o_ref[...] = jnp.maximum(_ln(s, g_ref[...], beta_ref[...]), 0.0)

        if nfull >= MGRID_H:
            compute()
        else:
            pid = pl.program_id(0)

            @pl.when(pid < nfull)
            def _():
                compute()

            @pl.when(pid >= nfull)
            def _():
                o_ref[...] = jnp.zeros_like(o_ref)

    return _msg_body


def _make_message(nfull, blk_ofs):
    return pl.pallas_call(
        _make_msg_body(nfull),
        grid=(MGRID_H,),
        in_specs=[
            pl.BlockSpec((MBLK, HID), lambda i: (i, 0)),
            pl.BlockSpec((MBLK, HID), lambda i: (i, 0)),
            pl.BlockSpec((MBLK, HID), lambda i: (i + blk_ofs, 0)),
            pl.BlockSpec((HID, HID), lambda i: (0, 0)),
            pl.BlockSpec((1, HID), lambda i: (0, 0)),
            pl.BlockSpec((1, HID), lambda i: (0, 0)),
            pl.BlockSpec((1, HID), lambda i: (0, 0)),
        ],
        out_specs=pl.BlockSpec((MBLK, HID), lambda i: (i, 0)),
        out_shape=jax.ShapeDtypeStruct((EH, HID), jnp.float32),
    )


_MESSAGE_A = _make_message(MGRID_H, 0)
_MESSAGE_B = _make_message(NFULL1, MGRID_H)


def _upd_body(h_ref, agga_ref, aggb_ref, g_ref, beta_ref, o_ref):
    t = (h_ref[...] + agga_ref[0] + agga_ref[1]
         + aggb_ref[0] + aggb_ref[1])
    o_ref[...] = jnp.maximum(_ln(t, g_ref[...], beta_ref[...]), 0.0)


def _update(h, agga, aggb, g, beta):
    return pl.pallas_call(
        _upd_body,
        out_shape=jax.ShapeDtypeStruct((N, HID), jnp.float32),
    )(h, agga, aggb, g, beta)


def _updp_body(h_ref, agga_ref, aggb_ref, g_ref, beta_ref, wa_ref, wb_ref,
               o_ref, h1_ref, h2_ref):
    t = (h_ref[...] + agga_ref[0] + agga_ref[1]
         + aggb_ref[0] + aggb_ref[1])
    hn = jnp.maximum(_ln(t, g_ref[...], beta_ref[...]), 0.0)
    o_ref[...] = hn
    h1_ref[...] = jnp.dot(hn, wa_ref[...], preferred_element_type=jnp.float32)
    h2_ref[...] = jnp.dot(hn, wb_ref[...], preferred_element_type=jnp.float32)


def _update_proj(h, agga, aggb, g, beta, wa, wb):
    return pl.pallas_call(
        _updp_body,
        out_shape=(jax.ShapeDtypeStruct((N, HID), jnp.float32),
                   jax.ShapeDtypeStruct((N, HID), jnp.float32),
                   jax.ShapeDtypeStruct((N, HID), jnp.float32)),
    )(h, agga, aggb, g, beta, wa, wb)


def _readout_body(h_ref, attw_ref, attb_ref, batch_ref, sg_ref, sgemb_ref,
                  w1t_ref, w1b_ref, b1_ref, w2_ref, b2_ref, o_ref):
    h = h_ref[...]
    s = jnp.dot(h, attw_ref[...], preferred_element_type=jnp.float32) + attb_ref[...]
    m = jnp.max(s)
    e = jnp.exp(s - m)
    alpha = e / jnp.sum(e)
    gi = lax.broadcasted_iota(jnp.int32, (NG, N), 0)
    oh = (gi == batch_ref[...]).astype(jnp.float32)
    pooled = jnp.dot(oh, alpha * h, preferred_element_type=jnp.float32)
    sgi = lax.broadcasted_iota(jnp.int32, (NG, SGN), 1)
    ohsg = (sgi == sg_ref[...]).astype(jnp.float32)
    sg = jnp.dot(ohsg, sgemb_ref[...], preferred_element_type=jnp.float32)
    hh = (jnp.dot(pooled, w1t_ref[...], preferred_element_type=jnp.float32)
          + jnp.dot(sg, w1b_ref[...], preferred_element_type=jnp.float32)
          + b1_ref[...])
    hh = jnp.maximum(hh, 0.0)
    o_ref[...] = jnp.dot(hh, w2_ref[...], preferred_element_type=jnp.float32) + b2_ref[...]


def _readout(h, attw, attb, batch2d, sg2d, sgemb, w1t, w1b, b1, w2, b2):
    return pl.pallas_call(
        _readout_body,
        out_shape=jax.ShapeDtypeStruct((NG, HID), jnp.float32),
    )(h, attw, attb, batch2d, sg2d, sgemb, w1t, w1b, b1, w2, b2)


SUP = 2
NSUPW = CPH // SUP


EPS = EH // NS
CPS = EPS // CHUNK


def _gather_body(h1, h2, ridx, cidx, g1, g2, idx_v, bufa, bufb,
                 gsa, gsb, ssa, ssb, spm_tab):
    c = lax.axis_index("c")
    s = lax.axis_index("s")
    bufs = (bufa, bufb)
    gs = (gsa, gsb)
    ss = (ssa, ssb)

    def run(tab_hbm, idx_hbm, out_hbm):
        base = s * RS

        @pl.when(s < NS - 1)
        def _():
            pltpu.sync_copy(tab_hbm.at[pl.ds(base, RS)], spm_tab.at[pl.ds(base, RS)])

        @pl.when(s == NS - 1)
        def _():
            pltpu.sync_copy(tab_hbm.at[pl.ds((NS - 1) * RS, RS_LAST)],
                            spm_tab.at[pl.ds((NS - 1) * RS, RS_LAST)])

        pltpu.sync_copy(idx_hbm.at[pl.ds(s * EPS, EPS)], idx_v)
        plsc.subcore_barrier()

        def fire_gather(j, p):
            pltpu.async_copy(spm_tab.at[idx_v.at[pl.ds(j * CHUNK, CHUNK)]],
                             bufs[p], gs[p])

        def wait_gather(j, p):
            pltpu.make_async_copy(spm_tab.at[idx_v.at[pl.ds(j * CHUNK, CHUNK)]],
                                  bufs[p], gs[p]).wait()

        def wait_store(p):
            pltpu.make_async_copy(bufs[p], out_hbm.at[pl.ds(0, CHUNK)], ss[p]).wait()

        fire_gather(0, 0)

        def body(jj, _):
            for p in range(2):
                j = jj * 2 + p
                q = 1 - p

                @pl.when(j + 1 < CPS)
                def _():
                    @pl.when(j >= 1)
                    def _():
                        wait_store(q)
                    fire_gather(j + 1, q)

                wait_gather(j, p)
                pltpu.async_copy(bufs[p], out_hbm.at[pl.ds(s * EPS + j * CHUNK, CHUNK)],
                                 ss[p])
            return 0

        lax.fori_loop(0, CPS // 2, body, 0)
        wait_store(0)
        wait_store(1)

    @pl.when(c == 0)
    def _():
        run(h1, ridx, g1)

    @pl.when(c == 1)
    def _():
        run(h2, cidx, g2)


_GATHER_K = None


def _gather_sc(h1, h2, ridx_flat, cidx_flat):
    global _GATHER_K
    if _GATHER_K is None:
        _GATHER_K = pl.kernel(
            _gather_body,
            out_type=(jax.ShapeDtypeStruct((EH, HID), jnp.float32),
                      jax.ShapeDtypeStruct((EH, HID), jnp.float32)),
            mesh=_sc_mesh(),
            scratch_types=(
                [pltpu.VMEM((EPS,), jnp.int32)]
                + [pltpu.VMEM((CHUNK, HID), jnp.float32)] * 2
                + [pltpu.SemaphoreType.DMA] * 4
                + [pltpu.VMEM_SHARED((N, HID), jnp.float32)]
            ),
        )
    return _GATHER_K(h1, h2, ridx_flat, cidx_flat)


def _scatter_body(msg, ridx, zeros_n, out, ridx_v, mbufa, mbufb,
                  lsa, lsb, asa, asb, acc):
    c = lax.axis_index("c")
    s = lax.axis_index("s")
    wid = s * NC + c
    pltpu.sync_copy(ridx.at[pl.ds(wid * CPH, CPH)], ridx_v)
    base = s * RS

    @pl.when(s < NS - 1)
    def _():
        pltpu.sync_copy(zeros_n.at[pl.ds(base, RS)], acc.at[pl.ds(base, RS)])

    @pl.when(s == NS - 1)
    def _():
        pltpu.sync_copy(zeros_n.at[pl.ds((NS - 1) * RS, RS_LAST)],
                        acc.at[pl.ds((NS - 1) * RS, RS_LAST)])

    plsc.subcore_barrier()

    mbufs = (mbufa, mbufb)
    lsem = (lsa, lsb)
    asem = (asa, asb)

    def fire_load(j, p):
        chunk = wid * CPH + j
        pltpu.async_copy(msg.at[pl.ds(chunk * CHUNK, CHUNK)], mbufs[p], lsem[p])

    def wait_add(p):
        pltpu.make_async_copy(mbufs[p], acc.at[pl.ds(0, CHUNK)], asem[p]).wait()

    fire_load(0, 0)

    def body(jj, _):
        for p in range(2):
            j = jj * 2 + p
            q = 1 - p

            @pl.when(j + 1 < CPH)
            def _():
                @pl.when(j >= 1)
                def _():
                    wait_add(q)
                fire_load(j + 1, q)

            pltpu.make_async_copy(msg.at[pl.ds(0, CHUNK)], mbufs[p], lsem[p]).wait()
            pltpu.async_copy(mbufs[p], acc.at[ridx_v.at[j]], asem[p], add=True)
        return 0

    lax.fori_loop(0, CPH // 2, body, 0)
    wait_add(0)
    wait_add(1)
    plsc.subcore_barrier()

    @pl.when(s < NS - 1)
    def _():
        pltpu.sync_copy(acc.at[pl.ds(base, RS)], out.at[c, pl.ds(base, RS)])

    @pl.when(s == NS - 1)
    def _():
        pltpu.sync_copy(acc.at[pl.ds((NS - 1) * RS, RS_LAST)],
                        out.at[c, pl.ds((NS - 1) * RS, RS_LAST)])


_SCATTER_K = None


def _scatter_sc(msg, ridx, zeros_n):
    global _SCATTER_K
    if _SCATTER_K is None:
        _SCATTER_K = pl.kernel(
            _scatter_body,
            out_type=jax.ShapeDtypeStruct((NC, N, HID), jnp.float32),
            mesh=_sc_mesh(),
            scratch_types=(
                [pltpu.VMEM((CPH, CHUNK), jnp.int32)]
                + [pltpu.VMEM((CHUNK, HID), jnp.float32)] * 2
                + [pltpu.SemaphoreType.DMA] * 4
                + [pltpu.VMEM_SHARED((N, HID), jnp.float32)]
            ),
        )
    return _SCATTER_K(msg, ridx, zeros_n)


def kernel(params, x, edge_attr, edge_index, batch, space_group):
    p = params
    f32 = jnp.float32
    row = edge_index[0]
    col = edge_index[1]
    pad_idx = (jnp.arange(EPAD - E, dtype=jnp.int32) % N)
    row_p = jnp.concatenate([row, pad_idx]).reshape(NCHUNK, CHUNK)
    col_p = jnp.concatenate([col, pad_idx]).reshape(NCHUNK, CHUNK)
    ea_pad = jnp.concatenate([edge_attr, jnp.zeros((EPAD - E, 3), f32)])
    zeros_n = jnp.zeros((N, HID), f32)

    def v2(a):
        return a.reshape(1, -1)

    h = _encode_nodes(x, p['node_w'], v2(p['node_b']), v2(p['node_g']), v2(p['node_beta']))
    ea_enc = _encode_edges(ea_pad, p['edge_w'], v2(p['edge_b']), v2(p['edge_g']), v2(p['edge_beta']))

    row_a, row_b = row_p[:NCH], row_p[NCH:]
    row_fa, row_fb = row_a.reshape(-1), row_b.reshape(-1)
    col_fa = col_p[:NCH].reshape(-1)
    col_fb = col_p[NCH:].reshape(-1)

    h1 = h2 = None
    for l in range(DEPTH):
        W = p['mp_w'][l]
        wa, wb, wc = W[:HID], W[HID:2 * HID], W[2 * HID:]
        mb, mg, mbeta = v2(p['mp_b'][l]), v2(p['mp_g'][l]), v2(p['mp_beta'][l])
        if l == 0:
            h1, h2 = _project(h, wa, wb)
        g1a, g2a = _gather_sc(h1, h2, row_fa, col_fa)
        row_fb2, _ = lax.optimization_barrier((row_fb, g1a))
        g1b, g2b = _gather_sc(h1, h2, row_fb2, col_fb)
        msg_a = _MESSAGE_A(g1a, g2a, ea_enc, wc, mb, mg, mbeta)
        msg_a2, _ = lax.optimization_barrier((msg_a, g1b))
        agg_a = _scatter_sc(msg_a2, row_a, zeros_n)
        msg_b = _MESSAGE_B(g1b, g2b, ea_enc, wc, mb, mg, mbeta)
        msg_b2, _ = lax.optimization_barrier((msg_b, agg_a))
        agg_b = _scatter_sc(msg_b2, row_b, zeros_n)
        if l + 1 < DEPTH:
            Wn = p['mp_w'][l + 1]
            h, h1, h2 = _update_proj(h, agg_a, agg_b, v2(p['ln_g'][l]), v2(p['ln_b'][l]),
                                     Wn[:HID], Wn[HID:2 * HID])
        else:
            h = _update(h, agg_a, agg_b, v2(p['ln_g'][l]), v2(p['ln_b'][l]))

    heads = ['e', 'st', 'cs', 'mt']
    odims = [1, 3, 7, 3]
    w1t = jnp.concatenate([p[k + '_w1'][:HID] for k in heads], axis=1)
    w1b = jnp.concatenate([p[k + '_w1'][HID:] for k in heads], axis=1)
    b1 = jnp.concatenate([p[k + '_b1'] for k in heads]).reshape(1, 4 * HID)
    w2 = jnp.zeros((4 * HID, HID), f32)
    b2 = jnp.zeros((1, HID), f32)
    off = 0
    for i, k in enumerate(heads):
        w2 = w2.at[i * HID:(i + 1) * HID, off:off + odims[i]].set(p[k + '_w2'])
        b2 = b2.at[0, off:off + odims[i]].set(p[k + '_b2'])
        off += odims[i]

    out = _readout(h, p['att_w'], p['att_b'].reshape(1, 1),
                   batch.reshape(1, N), space_group.reshape(NG, 1).astype(jnp.int32),
                   p['sg_emb'], w1t, w1b, b1, w2, b2)
    return (out[:, :1], out[:, 1:4], out[:, 4:11], out[:, 11:14])
